# baseline (speedup 1.0000x reference)
import functools

import jax
import jax.numpy as jnp
from jax import lax
from jax.experimental import pallas as pl
from jax.experimental.pallas import tpu as pltpu
from jax.experimental.pallas import tpu_sc as plsc

B, F, OUT = 16384, 26, 64
N = B * F
NC, NS, L = 2, 16, 16
NW = NC * NS
PER_W = N // NW
CHUNK = 128
N_CHUNKS = PER_W // CHUNK


def _prep_body(u0_ref, u1_ref, u2_ref, u3_ref, p1_ref, p2_ref):
    p1_ref[...] = jnp.dot(u0_ref[...], u1_ref[...],
                          preferred_element_type=jnp.float32)
    p2_ref[...] = jnp.dot(u2_ref[...], u3_ref[...],
                          preferred_element_type=jnp.float32)


def _expanded_tables(U0, U1, U2, U3):
    p1, p2 = pl.pallas_call(
        _prep_body,
        out_shape=(
            jax.ShapeDtypeStruct((160, 800), jnp.float32),
            jax.ShapeDtypeStruct((640, 50), jnp.float32),
        ),
    )(U0.reshape(160, 8), U1.reshape(8, 800),
      U2.reshape(640, 8), U3.reshape(8, 50))
    out1 = (p1.reshape(40, 4, 25, 4, 8)
            .transpose(0, 2, 1, 3, 4)
            .reshape(1000, 16, 8))
    out2 = (p2.reshape(8, 40, 2, 25, 2)
            .transpose(0, 1, 3, 2, 4)
            .reshape(8, 1000, 4))
    a1p = jnp.broadcast_to(
        out1.reshape(500, 2, 16, 8).transpose(0, 3, 1, 2)[..., None],
        (500, 8, 2, 16, 4)).reshape(500, 8, 128)
    a2x = jnp.broadcast_to(
        out2[:, :, None, :], (8, 1000, 32, 4)).reshape(8, 1000, 128)
    return a1p, a2x


def _ebuild_body(a1p_ref, a2x_ref, e_ref):
    acc = a2x_ref[0] * a1p_ref[0, 0][None, :]
    for k in range(1, 8):
        acc = acc + a2x_ref[k] * a1p_ref[0, k][None, :]
    e_ref[...] = acc


def _build_e(a1p, a2x):
    return pl.pallas_call(
        _ebuild_body,
        grid=(500,),
        in_specs=[
            pl.BlockSpec((1, 8, 128), lambda p: (p, 0, 0)),
            pl.BlockSpec((8, 1000, 128), lambda p: (0, 0, 0)),
        ],
        out_specs=pl.BlockSpec((1000, 128), lambda p: (p, 0)),
        out_shape=jax.ShapeDtypeStruct((500000, 128), jnp.float32),
    )(a1p, a2x)


def _sc_gather(x_flat, e2):
    mesh = plsc.VectorSubcoreMesh(
        core_axis_name="c", subcore_axis_name="s",
        num_cores=NC, num_subcores=NS)

    @functools.partial(
        pl.kernel, mesh=mesh,
        out_type=jax.ShapeDtypeStruct((N, 128), jnp.float32),
        scratch_types=[
            pltpu.VMEM((CHUNK,), jnp.int32),
            pltpu.VMEM((CHUNK,), jnp.int32),
            pltpu.VMEM((CHUNK, 128), jnp.float32),
            pltpu.SemaphoreType.DMA,
        ],
    )
    def k(x_hbm, e_hbm, out_hbm, idx_v, w_v, rows_v, sem):
        wid = lax.axis_index("s") * NC + lax.axis_index("c")
        w_base = wid * PER_W

        def chunk_body(g, _):
            base = w_base + g * CHUNK
            pltpu.sync_copy(x_hbm.at[pl.ds(base, CHUNK)], idx_v)

            def vec_body(j, _):
                v = idx_v[pl.ds(j * L, L)]
                r = (v.astype(jnp.float32) * jnp.float32(1e-3)
                     ).astype(jnp.int32)
                c = v - r * 1000
                big = c >= 1000
                r = jnp.where(big, r + 1, r)
                c = jnp.where(big, c - 1000, c)
                neg = c < 0
                r = jnp.where(neg, r - 1, r)
                c = jnp.where(neg, c + 1000, c)
                w_v[pl.ds(j * L, L)] = (r >> 1) * 1000 + c
                return _

            lax.fori_loop(0, CHUNK // L, vec_body, None, unroll=True)
            pltpu.async_copy(e_hbm.at[w_v], rows_v, sem).wait()
            pltpu.sync_copy(rows_v, out_hbm.at[pl.ds(base, CHUNK)])
            return _

        lax.fori_loop(0, N_CHUNKS, chunk_body, None)

    return k(x_flat, e2)


def _post_body(pad_ref, o_ref):
    o_ref[...] = pad_ref[:, :OUT]


def _post_select(outpad):
    nb = 1024
    return pl.pallas_call(
        _post_body,
        grid=(N // nb,),
        in_specs=[pl.BlockSpec((nb, 128), lambda i: (i, 0))],
        out_specs=pl.BlockSpec((nb, OUT), lambda i: (i, 0)),
        out_shape=jax.ShapeDtypeStruct((N, OUT), jnp.float32),
    )(outpad)


def kernel(x, U0, U1, U2, U3):
    x_flat = x.reshape(N)
    a1p, a2x = _expanded_tables(U0, U1, U2, U3)
    e2 = _build_e(a1p, a2x)
    outpad = _sc_gather(x_flat, e2)
    s = ((x_flat // 1000) & 1) == 1
    out = jnp.where(s[:, None], outpad[:, OUT:], outpad[:, :OUT])
    return out.reshape(B, F, OUT)


# speedup vs baseline: 1.1452x; 1.1452x over previous
import functools

import jax
import jax.numpy as jnp
from jax import lax
from jax.experimental import pallas as pl
from jax.experimental.pallas import tpu as pltpu
from jax.experimental.pallas import tpu_sc as plsc

B, F, OUT = 16384, 26, 64
N = B * F
NC, NS, L = 2, 16, 16
NW = NC * NS
PER_W = N // NW
CHUNK = 128
N_CHUNKS = PER_W // CHUNK


def _prep_body(u0_ref, u1_ref, u2_ref, u3_ref, p1_ref, p2_ref):
    p1_ref[...] = jnp.dot(u0_ref[...], u1_ref[...],
                          preferred_element_type=jnp.float32)
    p2_ref[...] = jnp.dot(u2_ref[...], u3_ref[...],
                          preferred_element_type=jnp.float32)


def _expanded_tables(U0, U1, U2, U3):
    p1, p2 = pl.pallas_call(
        _prep_body,
        out_shape=(
            jax.ShapeDtypeStruct((160, 800), jnp.float32),
            jax.ShapeDtypeStruct((640, 50), jnp.float32),
        ),
    )(U0.reshape(160, 8), U1.reshape(8, 800),
      U2.reshape(640, 8), U3.reshape(8, 50))
    out1 = (p1.reshape(40, 4, 25, 4, 8)
            .transpose(0, 2, 1, 3, 4)
            .reshape(1000, 16, 8))
    out2 = (p2.reshape(8, 40, 2, 25, 2)
            .transpose(0, 1, 3, 2, 4)
            .reshape(8, 1000, 4))
    a1p = jnp.broadcast_to(
        out1.reshape(500, 2, 16, 8).transpose(0, 3, 1, 2)[..., None],
        (500, 8, 2, 16, 4)).reshape(500, 8, 128)
    a2x = jnp.broadcast_to(
        out2[:, :, None, :], (8, 1000, 32, 4)).reshape(8, 1000, 128)
    return a1p, a2x


def _ebuild_body(a1p_ref, a2x_ref, e_ref, stash_ref):
    h = pl.program_id(1)

    @pl.when(h == 0)
    def _even():
        acc = a2x_ref[0] * a1p_ref[0, 0][None, :]
        for k in range(1, 8):
            acc = acc + a2x_ref[k] * a1p_ref[0, k][None, :]
        stash_ref[...] = acc
        e_ref[...] = acc

    @pl.when(h == 1)
    def _odd():
        acc = stash_ref[...]
        e_ref[...] = jnp.concatenate([acc[:, OUT:], acc[:, :OUT]], axis=1)


def _build_e(a1p, a2x):
    return pl.pallas_call(
        _ebuild_body,
        grid=(500, 2),
        in_specs=[
            pl.BlockSpec((1, 8, 128), lambda p, h: (p, 0, 0)),
            pl.BlockSpec((8, 1000, 128), lambda p, h: (0, 0, 0)),
        ],
        out_specs=pl.BlockSpec((1000, 128), lambda p, h: (h * 500 + p, 0)),
        out_shape=jax.ShapeDtypeStruct((1000000, 128), jnp.float32),
        scratch_shapes=[pltpu.VMEM((1000, 128), jnp.float32)],
    )(a1p, a2x)


def _sc_gather(x_flat, e2):
    mesh = plsc.VectorSubcoreMesh(
        core_axis_name="c", subcore_axis_name="s",
        num_cores=NC, num_subcores=NS)

    @functools.partial(
        pl.kernel, mesh=mesh,
        out_type=jax.ShapeDtypeStruct((N, 128), jnp.float32),
        scratch_types=[
            pltpu.VMEM((PER_W,), jnp.int32),
            pltpu.VMEM((PER_W,), jnp.int32),
            pltpu.VMEM((CHUNK, 128), jnp.float32),
            pltpu.VMEM((CHUNK, 128), jnp.float32),
            pltpu.SemaphoreType.DMA,
            pltpu.SemaphoreType.DMA,
        ],
    )
    def k(x_hbm, e_hbm, out_hbm, idx_v, w_v, r0_v, r1_v, sem0, sem1):
        wid = lax.axis_index("s") * NC + lax.axis_index("c")
        w_base = wid * PER_W
        pltpu.sync_copy(x_hbm.at[pl.ds(w_base, PER_W)], idx_v)

        def vec_body(j, _):
            v = idx_v[pl.ds(j * L, L)]
            r = (v.astype(jnp.float32) * jnp.float32(1e-3)
                 ).astype(jnp.int32)
            c = v - r * 1000
            big = c >= 1000
            r = jnp.where(big, r + 1, r)
            c = jnp.where(big, c - 1000, c)
            neg = c < 0
            r = jnp.where(neg, r - 1, r)
            c = jnp.where(neg, c + 1000, c)
            w_v[pl.ds(j * L, L)] = ((r >> 1) * 1000 + c
                                    + (r & 1) * 500000)
            return _

        lax.fori_loop(0, PER_W // L, vec_body, None)

        def fire(g, rows_v, sem):
            pltpu.async_copy(e_hbm.at[w_v.at[pl.ds(g * CHUNK, CHUNK)]],
                             rows_v, sem)

        def drain(g, rows_v, sem):
            pltpu.make_async_copy(
                e_hbm.at[w_v.at[pl.ds(g * CHUNK, CHUNK)]],
                rows_v, sem).wait()
            pltpu.sync_copy(rows_v,
                            out_hbm.at[pl.ds(w_base + g * CHUNK, CHUNK)])

        fire(0, r0_v, sem0)

        def pair_body(i, _):
            g0 = 2 * i
            fire(g0 + 1, r1_v, sem1)
            drain(g0, r0_v, sem0)

            @pl.when(g0 + 2 < N_CHUNKS)
            def _():
                fire(g0 + 2, r0_v, sem0)

            drain(g0 + 1, r1_v, sem1)
            return _

        lax.fori_loop(0, N_CHUNKS // 2, pair_body, None)

    return k(x_flat, e2)


def _post_body(pad_ref, o_ref):
    o_ref[...] = pad_ref[:, :OUT]


def _post_select(outpad):
    nb = 1024
    return pl.pallas_call(
        _post_body,
        grid=(N // nb,),
        in_specs=[pl.BlockSpec((nb, 128), lambda i: (i, 0))],
        out_specs=pl.BlockSpec((nb, OUT), lambda i: (i, 0)),
        out_shape=jax.ShapeDtypeStruct((N, OUT), jnp.float32),
    )(outpad)


def kernel(x, U0, U1, U2, U3):
    x_flat = x.reshape(N)
    a1p, a2x = _expanded_tables(U0, U1, U2, U3)
    e2 = _build_e(a1p, a2x)
    outpad = _sc_gather(x_flat, e2)
    return outpad[:, :OUT].reshape(B, F, OUT)


# speedup vs baseline: 1.2215x; 1.0667x over previous
import functools

import jax
import jax.numpy as jnp
from jax import lax
from jax.experimental import pallas as pl
from jax.experimental.pallas import tpu as pltpu
from jax.experimental.pallas import tpu_sc as plsc

B, F, OUT = 16384, 26, 64
N = B * F
NC, NS, L = 2, 16, 16
NW = NC * NS
PER_W = N // NW
CHUNK = 128
N_CHUNKS = PER_W // CHUNK


def _prep_body(u0_ref, u1_ref, u2_ref, u3_ref, p1_ref, p2_ref):
    p1_ref[...] = jnp.dot(u0_ref[...], u1_ref[...],
                          preferred_element_type=jnp.float32)
    p2_ref[...] = jnp.dot(u2_ref[...], u3_ref[...],
                          preferred_element_type=jnp.float32)


def _expanded_tables(U0, U1, U2, U3):
    p1, p2 = pl.pallas_call(
        _prep_body,
        out_shape=(
            jax.ShapeDtypeStruct((160, 800), jnp.float32),
            jax.ShapeDtypeStruct((640, 50), jnp.float32),
        ),
    )(U0.reshape(160, 8), U1.reshape(8, 800),
      U2.reshape(640, 8), U3.reshape(8, 50))
    out1 = (p1.reshape(40, 4, 25, 4, 8)
            .transpose(0, 2, 1, 3, 4)
            .reshape(1000, 16, 8))
    out2 = (p2.reshape(8, 40, 2, 25, 2)
            .transpose(0, 1, 3, 2, 4)
            .reshape(8, 1000, 4))
    a1p = jnp.broadcast_to(
        out1.reshape(500, 2, 16, 8).transpose(0, 3, 1, 2)[..., None],
        (500, 8, 2, 16, 4)).reshape(500, 8, 128)
    a2x = jnp.broadcast_to(
        out2[:, :, None, :], (8, 1000, 32, 4)).reshape(8, 1000, 128)
    return a1p, a2x


def _ebuild_body(a1p_ref, a2x_ref, e_ref):
    def rnd(f):
        w = jax.lax.bitcast_convert_type(f, jnp.int32)
        return w + 0x7FFF + ((w >> 16) & 1)

    packed = []
    for t in range(2):
        acc = a2x_ref[0] * a1p_ref[t, 0][None, :]
        for k in range(1, 8):
            acc = acc + a2x_ref[k] * a1p_ref[t, k][None, :]
        lo = jax.lax.shift_right_logical(rnd(acc[:, :OUT]), 16)
        hi = rnd(acc[:, OUT:]) & jnp.int32(-65536)
        packed.append(lo | hi)
    e_ref[...] = jnp.concatenate(packed, axis=1)


def _build_e(a1p, a2x):
    return pl.pallas_call(
        _ebuild_body,
        grid=(250,),
        in_specs=[
            pl.BlockSpec((2, 8, 128), lambda p: (p, 0, 0)),
            pl.BlockSpec((8, 1000, 128), lambda p: (0, 0, 0)),
        ],
        out_specs=pl.BlockSpec((1000, 128), lambda p: (p, 0)),
        out_shape=jax.ShapeDtypeStruct((250000, 128), jnp.int32),
    )(a1p, a2x)


def _sc_gather(x_flat, e2):
    mesh = plsc.VectorSubcoreMesh(
        core_axis_name="c", subcore_axis_name="s",
        num_cores=NC, num_subcores=NS)

    @functools.partial(
        pl.kernel, mesh=mesh,
        out_type=jax.ShapeDtypeStruct((N, 128), jnp.int32),
        scratch_types=[
            pltpu.VMEM((PER_W,), jnp.int32),
            pltpu.VMEM((PER_W,), jnp.int32),
            pltpu.VMEM((CHUNK, 128), jnp.int32),
            pltpu.VMEM((CHUNK, 128), jnp.int32),
            pltpu.SemaphoreType.DMA,
            pltpu.SemaphoreType.DMA,
        ],
    )
    def k(x_hbm, e_hbm, out_hbm, idx_v, w_v, r0_v, r1_v, sem0, sem1):
        wid = lax.axis_index("s") * NC + lax.axis_index("c")
        w_base = wid * PER_W
        pltpu.sync_copy(x_hbm.at[pl.ds(w_base, PER_W)], idx_v)

        def vec_body(j, _):
            v = idx_v[pl.ds(j * L, L)]
            r = (v.astype(jnp.float32) * jnp.float32(1e-3)
                 ).astype(jnp.int32)
            c = v - r * 1000
            big = c >= 1000
            r = jnp.where(big, r + 1, r)
            c = jnp.where(big, c - 1000, c)
            neg = c < 0
            r = jnp.where(neg, r - 1, r)
            c = jnp.where(neg, c + 1000, c)
            w_v[pl.ds(j * L, L)] = (r >> 2) * 1000 + c
            return _

        lax.fori_loop(0, PER_W // L, vec_body, None)

        def fire(g, rows_v, sem):
            pltpu.async_copy(e_hbm.at[w_v.at[pl.ds(g * CHUNK, CHUNK)]],
                             rows_v, sem)

        def drain(g, rows_v, sem):
            pltpu.make_async_copy(
                e_hbm.at[w_v.at[pl.ds(g * CHUNK, CHUNK)]],
                rows_v, sem).wait()
            pltpu.sync_copy(rows_v,
                            out_hbm.at[pl.ds(w_base + g * CHUNK, CHUNK)])

        fire(0, r0_v, sem0)

        def pair_body(i, _):
            g0 = 2 * i
            fire(g0 + 1, r1_v, sem1)
            drain(g0, r0_v, sem0)

            @pl.when(g0 + 2 < N_CHUNKS)
            def _():
                fire(g0 + 2, r0_v, sem0)

            drain(g0 + 1, r1_v, sem1)
            return _

        lax.fori_loop(0, N_CHUNKS // 2, pair_body, None)

    return k(x_flat, e2)


def _post_body(pad_ref, o_ref):
    o_ref[...] = pad_ref[:, :OUT]


def _post_select(outpad):
    nb = 1024
    return pl.pallas_call(
        _post_body,
        grid=(N // nb,),
        in_specs=[pl.BlockSpec((nb, 128), lambda i: (i, 0))],
        out_specs=pl.BlockSpec((nb, OUT), lambda i: (i, 0)),
        out_shape=jax.ShapeDtypeStruct((N, OUT), jnp.float32),
    )(outpad)


def kernel(x, U0, U1, U2, U3):
    x_flat = x.reshape(N)
    a1p, a2x = _expanded_tables(U0, U1, U2, U3)
    e2 = _build_e(a1p, a2x)
    outpad = _sc_gather(x_flat, e2)
    r = x_flat // 1000
    cw = jnp.where(((r >> 1) & 1)[:, None] == 1,
                   outpad[:, OUT:], outpad[:, :OUT])
    bits = jnp.where((r & 1)[:, None] == 1,
                     cw & jnp.int32(-65536),
                     jax.lax.shift_left(cw, 16))
    out = jax.lax.bitcast_convert_type(bits, jnp.float32)
    return out.reshape(B, F, OUT)


# speedup vs baseline: 1.4771x; 1.2093x over previous
import functools

import jax
import jax.numpy as jnp
from jax import lax
from jax.experimental import pallas as pl
from jax.experimental.pallas import tpu as pltpu
from jax.experimental.pallas import tpu_sc as plsc

B, F, OUT = 16384, 26, 64
N = B * F
NC, NS, L = 2, 16, 16
NW = NC * NS
PER_W = N // NW
CHUNK_B = 4
CHUNK = CHUNK_B * F
N_CHUNKS = PER_W // CHUNK


def _prep_body(u0_ref, u1_ref, u2_ref, u3_ref, p1_ref, p2_ref):
    p1_ref[...] = jnp.dot(u0_ref[...], u1_ref[...],
                          preferred_element_type=jnp.float32)
    p2_ref[...] = jnp.dot(u2_ref[...], u3_ref[...],
                          preferred_element_type=jnp.float32)


def _expanded_tables(U0, U1, U2, U3):
    p1, p2 = pl.pallas_call(
        _prep_body,
        out_shape=(
            jax.ShapeDtypeStruct((160, 800), jnp.float32),
            jax.ShapeDtypeStruct((640, 50), jnp.float32),
        ),
    )(U0.reshape(160, 8), U1.reshape(8, 800),
      U2.reshape(640, 8), U3.reshape(8, 50))
    out1 = (p1.reshape(40, 4, 25, 4, 8)
            .transpose(0, 2, 1, 3, 4)
            .reshape(1000, 16, 8))
    out2 = (p2.reshape(8, 40, 2, 25, 2)
            .transpose(0, 1, 3, 2, 4)
            .reshape(8, 1000, 4))
    a1p = jnp.broadcast_to(
        out1.reshape(500, 2, 16, 8).transpose(0, 3, 1, 2)[..., None],
        (500, 8, 2, 16, 4)).reshape(500, 8, 128)
    a2x = jnp.broadcast_to(
        out2[:, :, None, :], (8, 1000, 32, 4)).reshape(8, 1000, 128)
    return a1p, a2x


def _ebuild_body(a1p_ref, a2x_ref, e_ref):
    def rnd(f):
        w = jax.lax.bitcast_convert_type(f, jnp.int32)
        return w + 0x7FFF + ((w >> 16) & 1)

    packed = []
    for t in range(2):
        acc = a2x_ref[0] * a1p_ref[t, 0][None, :]
        for k in range(1, 8):
            acc = acc + a2x_ref[k] * a1p_ref[t, k][None, :]
        lo = jax.lax.shift_right_logical(rnd(acc[:, :OUT]), 16)
        hi = rnd(acc[:, OUT:]) & jnp.int32(-65536)
        packed.append(lo | hi)
    e_ref[...] = jnp.concatenate(packed, axis=1)


def _build_e(a1p, a2x):
    return pl.pallas_call(
        _ebuild_body,
        grid=(250,),
        in_specs=[
            pl.BlockSpec((2, 8, 128), lambda p: (p, 0, 0)),
            pl.BlockSpec((8, 1000, 128), lambda p: (0, 0, 0)),
        ],
        out_specs=pl.BlockSpec((1000, 128), lambda p: (p, 0)),
        out_shape=jax.ShapeDtypeStruct((250000, 128), jnp.int32),
    )(a1p, a2x)


def _sc_gather(x_flat, e2):
    mesh = plsc.VectorSubcoreMesh(
        core_axis_name="c", subcore_axis_name="s",
        num_cores=NC, num_subcores=NS)

    @functools.partial(
        pl.kernel, mesh=mesh,
        out_type=jax.ShapeDtypeStruct((B, 32, 128), jnp.int32),
        scratch_types=[
            pltpu.VMEM((PER_W,), jnp.int32),
            pltpu.VMEM((PER_W,), jnp.int32),
            pltpu.VMEM((CHUNK + 32, 128), jnp.int32),
            pltpu.VMEM((CHUNK + 32, 128), jnp.int32),
            pltpu.SemaphoreType.DMA,
            pltpu.SemaphoreType.DMA,
        ],
    )
    def k(x_hbm, e_hbm, out_hbm, idx_v, w_v, r0_v, r1_v, sem0, sem1):
        wid = lax.axis_index("s") * NC + lax.axis_index("c")
        w_base = wid * PER_W
        b_base = wid * (PER_W // F)
        pltpu.sync_copy(x_hbm.at[pl.ds(w_base, PER_W)], idx_v)

        def vec_body(j, _):
            v = idx_v[pl.ds(j * L, L)]
            r = (v.astype(jnp.float32) * jnp.float32(1e-3)
                 ).astype(jnp.int32)
            c = v - r * 1000
            big = c >= 1000
            r = jnp.where(big, r + 1, r)
            c = jnp.where(big, c - 1000, c)
            neg = c < 0
            r = jnp.where(neg, r - 1, r)
            c = jnp.where(neg, c + 1000, c)
            w_v[pl.ds(j * L, L)] = (r >> 2) * 1000 + c
            return _

        lax.fori_loop(0, PER_W // L, vec_body, None)

        def fire(g, rows_v, sem):
            pltpu.async_copy(e_hbm.at[w_v.at[pl.ds(g * CHUNK, CHUNK)]],
                             rows_v.at[pl.ds(0, CHUNK)], sem)

        def drain(g, rows_v, sem):
            pltpu.make_async_copy(
                e_hbm.at[w_v.at[pl.ds(g * CHUNK, CHUNK)]],
                rows_v.at[pl.ds(0, CHUNK)], sem).wait()
            for k in range(CHUNK_B):
                pltpu.sync_copy(rows_v.at[pl.ds(k * F, 32)],
                                out_hbm.at[b_base + g * CHUNK_B + k])

        fire(0, r0_v, sem0)

        def pair_body(i, _):
            g0 = 2 * i
            fire(g0 + 1, r1_v, sem1)
            drain(g0, r0_v, sem0)

            @pl.when(g0 + 2 < N_CHUNKS)
            def _():
                fire(g0 + 2, r0_v, sem0)

            drain(g0 + 1, r1_v, sem1)
            return _

        lax.fori_loop(0, N_CHUNKS // 2, pair_body, None)

    return k(x_flat, e2)


def _post_body(pad_ref, o_ref):
    o_ref[...] = pad_ref[:, :OUT]


def _post_select(outpad):
    nb = 1024
    return pl.pallas_call(
        _post_body,
        grid=(N // nb,),
        in_specs=[pl.BlockSpec((nb, 128), lambda i: (i, 0))],
        out_specs=pl.BlockSpec((nb, OUT), lambda i: (i, 0)),
        out_shape=jax.ShapeDtypeStruct((N, OUT), jnp.float32),
    )(outpad)


def kernel(x, U0, U1, U2, U3):
    x_flat = x.reshape(N)
    a1p, a2x = _expanded_tables(U0, U1, U2, U3)
    e2 = _build_e(a1p, a2x)
    out3 = _sc_gather(x_flat, e2)
    r = x // 1000
    cw = jnp.where(((r >> 1) & 1)[:, :, None] == 1,
                   out3[:, :F, OUT:], out3[:, :F, :OUT])
    bits = jnp.where((r & 1)[:, :, None] == 1,
                     cw & jnp.int32(-65536),
                     jax.lax.shift_left(cw, 16))
    return jax.lax.bitcast_convert_type(bits, jnp.float32)


# speedup vs baseline: 1.5020x; 1.0168x over previous
import functools

import jax
import jax.numpy as jnp
from jax import lax
from jax.experimental import pallas as pl
from jax.experimental.pallas import tpu as pltpu
from jax.experimental.pallas import tpu_sc as plsc

B, F, OUT = 16384, 26, 64
N = B * F
NC, NS, L = 2, 16, 16
NW = NC * NS
PER_W = N // NW
CHUNK_B = 4
CHUNK = CHUNK_B * F
N_CHUNKS = PER_W // CHUNK


def _prep_body(u0_ref, u1_ref, u2_ref, u3_ref, p1_ref, p2_ref):
    p1_ref[...] = jnp.dot(u0_ref[...], u1_ref[...],
                          preferred_element_type=jnp.float32)
    p2_ref[...] = jnp.dot(u2_ref[...], u3_ref[...],
                          preferred_element_type=jnp.float32)


def _expanded_tables(U0, U1, U2, U3):
    p1, p2 = pl.pallas_call(
        _prep_body,
        out_shape=(
            jax.ShapeDtypeStruct((160, 800), jnp.float32),
            jax.ShapeDtypeStruct((640, 50), jnp.float32),
        ),
    )(U0.reshape(160, 8), U1.reshape(8, 800),
      U2.reshape(640, 8), U3.reshape(8, 50))
    out1 = (p1.reshape(40, 4, 25, 4, 8)
            .transpose(0, 2, 1, 3, 4)
            .reshape(1000, 16, 8))
    out2 = (p2.reshape(8, 40, 2, 25, 2)
            .transpose(0, 1, 3, 2, 4)
            .reshape(8, 1000, 4))
    a1p = jnp.broadcast_to(
        out1.reshape(500, 2, 16, 8).transpose(0, 3, 1, 2)[..., None],
        (500, 8, 2, 16, 4)).reshape(500, 8, 128)
    a2x = jnp.broadcast_to(
        out2[:, :, None, :], (8, 1000, 32, 4)).reshape(8, 1000, 128)
    return a1p, a2x


def _ebuild_body(a1p_ref, a2x_ref, e_ref):
    def rnd(f):
        w = jax.lax.bitcast_convert_type(f, jnp.int32)
        return w + 0x7FFF + ((w >> 16) & 1)

    packed = []
    for t in range(4):
        acc = a2x_ref[0] * a1p_ref[t, 0][None, :]
        for k in range(1, 8):
            acc = acc + a2x_ref[k] * a1p_ref[t, k][None, :]
        lo = jax.lax.shift_right_logical(rnd(acc[:, :OUT]), 16)
        hi = rnd(acc[:, OUT:]) & jnp.int32(-65536)
        packed.append(lo | hi)
    e_ref[...] = jnp.concatenate(
        [jnp.concatenate(packed[:2], axis=1),
         jnp.concatenate(packed[2:], axis=1)], axis=0)


def _build_e(a1p, a2x):
    return pl.pallas_call(
        _ebuild_body,
        grid=(125,),
        in_specs=[
            pl.BlockSpec((4, 8, 128), lambda p: (p, 0, 0)),
            pl.BlockSpec((8, 1000, 128), lambda p: (0, 0, 0)),
        ],
        out_specs=pl.BlockSpec((2000, 128), lambda p: (p, 0)),
        out_shape=jax.ShapeDtypeStruct((250000, 128), jnp.int32),
    )(a1p, a2x)


def _sc_gather(x_flat, e2):
    mesh = plsc.VectorSubcoreMesh(
        core_axis_name="c", subcore_axis_name="s",
        num_cores=NC, num_subcores=NS)

    @functools.partial(
        pl.kernel, mesh=mesh,
        out_type=jax.ShapeDtypeStruct((B, 32, 128), jnp.int32),
        scratch_types=[
            pltpu.VMEM((PER_W,), jnp.int32),
            pltpu.VMEM((PER_W,), jnp.int32),
            pltpu.VMEM((CHUNK + 32, 128), jnp.int32),
            pltpu.VMEM((CHUNK + 32, 128), jnp.int32),
            pltpu.SemaphoreType.DMA,
            pltpu.SemaphoreType.DMA,
        ],
    )
    def k(x_hbm, e_hbm, out_hbm, idx_v, w_v, r0_v, r1_v, sem0, sem1):
        wid = lax.axis_index("s") * NC + lax.axis_index("c")
        w_base = wid * PER_W
        b_base = wid * (PER_W // F)
        pltpu.sync_copy(x_hbm.at[pl.ds(w_base, PER_W)], idx_v)

        def vec_body(j, _):
            v = idx_v[pl.ds(j * L, L)]
            r = (v.astype(jnp.float32) * jnp.float32(1e-3)
                 ).astype(jnp.int32)
            c = v - r * 1000
            big = c >= 1000
            r = jnp.where(big, r + 1, r)
            c = jnp.where(big, c - 1000, c)
            neg = c < 0
            r = jnp.where(neg, r - 1, r)
            c = jnp.where(neg, c + 1000, c)
            w_v[pl.ds(j * L, L)] = (r >> 2) * 1000 + c
            return _

        lax.fori_loop(0, PER_W // L, vec_body, None)

        def fire(g, rows_v, sem):
            pltpu.async_copy(e_hbm.at[w_v.at[pl.ds(g * CHUNK, CHUNK)]],
                             rows_v.at[pl.ds(0, CHUNK)], sem)

        def drain(g, rows_v, sem):
            pltpu.make_async_copy(
                e_hbm.at[w_v.at[pl.ds(g * CHUNK, CHUNK)]],
                rows_v.at[pl.ds(0, CHUNK)], sem).wait()
            for k in range(CHUNK_B):
                pltpu.sync_copy(rows_v.at[pl.ds(k * F, 32)],
                                out_hbm.at[b_base + g * CHUNK_B + k])

        fire(0, r0_v, sem0)

        def pair_body(i, _):
            g0 = 2 * i
            fire(g0 + 1, r1_v, sem1)
            drain(g0, r0_v, sem0)

            @pl.when(g0 + 2 < N_CHUNKS)
            def _():
                fire(g0 + 2, r0_v, sem0)

            drain(g0 + 1, r1_v, sem1)
            return _

        lax.fori_loop(0, N_CHUNKS // 2, pair_body, None)

    return k(x_flat, e2)


def _post_body(pad_ref, o_ref):
    o_ref[...] = pad_ref[:, :OUT]


def _post_select(outpad):
    nb = 1024
    return pl.pallas_call(
        _post_body,
        grid=(N // nb,),
        in_specs=[pl.BlockSpec((nb, 128), lambda i: (i, 0))],
        out_specs=pl.BlockSpec((nb, OUT), lambda i: (i, 0)),
        out_shape=jax.ShapeDtypeStruct((N, OUT), jnp.float32),
    )(outpad)


def kernel(x, U0, U1, U2, U3):
    x_flat = x.reshape(N)
    a1p, a2x = _expanded_tables(U0, U1, U2, U3)
    e2 = _build_e(a1p, a2x)
    out3 = _sc_gather(x_flat, e2)
    r = x // 1000
    cw = jnp.where(((r >> 1) & 1)[:, :, None] == 1,
                   out3[:, :F, OUT:], out3[:, :F, :OUT])
    bits = jnp.where((r & 1)[:, :, None] == 1,
                     cw & jnp.int32(-65536),
                     jax.lax.shift_left(cw, 16))
    return jax.lax.bitcast_convert_type(bits, jnp.float32)
